# Initial kernel scaffold; baseline (speedup 1.0000x reference)
import jax
import jax.numpy as jnp
from jax.experimental import pallas as pl


def kernel(z, embedding):
    raise NotImplementedError("write your pallas kernel here")


# speedup vs baseline: 10.5297x; 10.5297x over previous
import functools

import jax
import jax.numpy as jnp
from jax.experimental import pallas as pl
from jax.experimental.pallas import tpu as pltpu
from jax.experimental.pallas import tpu_sc as plsc

NUM_E = 8192
DIM = 32
DPAD = 128
TOK = 32768
NB = 8
HW = 4096


@functools.cache
def _sc_gather():
    info = plsc.get_sparse_core_info()
    nw = info.num_cores * info.num_subcores
    b_per_w = TOK // nw
    n_chunk = 128
    mesh = plsc.VectorSubcoreMesh(core_axis_name="c", subcore_axis_name="s")

    @functools.partial(
        pl.kernel, mesh=mesh,
        out_type=jax.ShapeDtypeStruct((TOK, DPAD), jnp.float32),
        scratch_types=[
            pltpu.VMEM((b_per_w,), jnp.int32),
            pltpu.VMEM((2, n_chunk, DPAD), jnp.float32),
            pltpu.SemaphoreType.DMA,
            pltpu.SemaphoreType.DMA,
        ],
    )
    def gather(table_hbm, idx_hbm, out_hbm, idx_v, rows_v, sem_g, sem_s):
        wid = jax.lax.axis_index("s") * info.num_cores + jax.lax.axis_index("c")
        base = wid * b_per_w
        pltpu.sync_copy(idx_hbm.at[pl.ds(base, b_per_w)], idx_v)
        nchunks = b_per_w // n_chunk
        copies = [None, None]
        stores = [None, None]
        for j in range(nchunks):
            s = j % 2
            if stores[s] is not None:
                stores[s].wait()
            copies[s] = pltpu.async_copy(
                table_hbm.at[idx_v.at[pl.ds(j * n_chunk, n_chunk)]],
                rows_v.at[s], sem_g)
            copies[s].wait()
            stores[s] = pltpu.async_copy(
                rows_v.at[s], out_hbm.at[pl.ds(base + j * n_chunk, n_chunk)],
                sem_s)
        for st in stores:
            if st is not None:
                st.wait()

    return gather


def _st_loss_body(q_ref, z_ref, out_ref, loss_ref):
    q = q_ref[0, :, :DIM]
    zt = z_ref[0]
    qt = jax.lax.transpose(q, (1, 0))
    d = qt - zt
    out_ref[0] = zt + d
    part = jnp.sum(d * d, keepdims=True).reshape(1, 1)

    @pl.when(pl.program_id(0) == 0)
    def _():
        loss_ref[...] = jnp.zeros((1, 1), jnp.float32)

    loss_ref[...] += part


def kernel(z, embedding):
    z_perm = jnp.transpose(z, (0, 2, 3, 1))
    flat_z = z_perm.reshape(-1, DIM)
    distances = (jnp.sum(flat_z ** 2, axis=1, keepdims=True)
                 - 2.0 * flat_z @ embedding
                 + jnp.sum(embedding ** 2, axis=0, keepdims=True))
    idx = jnp.argmin(distances, axis=1)

    idx_c = jnp.clip(idx.astype(jnp.int32), 0, NUM_E - 1)
    table = jnp.pad(embedding.T, ((0, 0), (0, DPAD - DIM)))
    q_flat = _sc_gather()(table, idx_c)

    q3 = q_flat.reshape(NB, HW, DPAD)
    z3 = z.reshape(NB, DIM, HW)
    qst, loss_sum = pl.pallas_call(
        _st_loss_body,
        grid=(NB,),
        in_specs=[pl.BlockSpec((1, HW, DPAD), lambda i: (i, 0, 0)),
                  pl.BlockSpec((1, DIM, HW), lambda i: (i, 0, 0))],
        out_specs=[pl.BlockSpec((1, DIM, HW), lambda i: (i, 0, 0)),
                   pl.BlockSpec((1, 1), lambda i: (0, 0))],
        out_shape=[jax.ShapeDtypeStruct((NB, DIM, HW), jnp.float32),
                   jax.ShapeDtypeStruct((1, 1), jnp.float32)],
    )(q3, z3)
    quantized_st = qst.reshape(NB, DIM, 64, 64)
    loss = loss_sum[0, 0] * (1.25 / (TOK * DIM))
    return quantized_st, loss


# speedup vs baseline: 10.5327x; 1.0003x over previous
import functools

import jax
import jax.numpy as jnp
from jax.experimental import pallas as pl
from jax.experimental.pallas import tpu as pltpu
from jax.experimental.pallas import tpu_sc as plsc

NUM_E = 8192
DIM = 32
DPAD = 128
TOK = 32768
NB = 8
HW = 4096


@functools.cache
def _sc_gather():
    info = plsc.get_sparse_core_info()
    nw = info.num_cores * info.num_subcores
    b_per_w = TOK // nw
    n_chunk = 128
    mesh = plsc.VectorSubcoreMesh(core_axis_name="c", subcore_axis_name="s")

    @functools.partial(
        pl.kernel, mesh=mesh,
        out_type=jax.ShapeDtypeStruct((TOK, DPAD), jnp.float32),
        scratch_types=[
            pltpu.VMEM((b_per_w,), jnp.int32),
            pltpu.VMEM((2, n_chunk, DPAD), jnp.float32),
            pltpu.SemaphoreType.DMA,
            pltpu.SemaphoreType.DMA,
        ],
    )
    def gather(table_hbm, idx_hbm, out_hbm, idx_v, rows_v, sem_g, sem_s):
        wid = jax.lax.axis_index("s") * info.num_cores + jax.lax.axis_index("c")
        base = wid * b_per_w
        pltpu.sync_copy(idx_hbm.at[pl.ds(base, b_per_w)], idx_v)
        nchunks = b_per_w // n_chunk

        def start_gather(j, s):
            return pltpu.async_copy(
                table_hbm.at[idx_v.at[pl.ds(j * n_chunk, n_chunk)]],
                rows_v.at[s], sem_g)

        copies = [start_gather(0, 0), None]
        stores = [None, None]
        for j in range(nchunks):
            s = j % 2
            sn = (j + 1) % 2
            if j + 1 < nchunks:
                if stores[sn] is not None:
                    stores[sn].wait()
                copies[sn] = start_gather(j + 1, sn)
            copies[s].wait()
            stores[s] = pltpu.async_copy(
                rows_v.at[s], out_hbm.at[pl.ds(base + j * n_chunk, n_chunk)],
                sem_s)
        for st in stores:
            if st is not None:
                st.wait()

    return gather


def _st_loss_body(q_ref, z_ref, out_ref, loss_ref):
    q = q_ref[0, :, :DIM]
    zt = z_ref[0]
    qt = jax.lax.transpose(q, (1, 0))
    d = qt - zt
    out_ref[0] = zt + d
    part = jnp.sum(d * d, keepdims=True).reshape(1, 1)

    @pl.when(pl.program_id(0) == 0)
    def _():
        loss_ref[...] = jnp.zeros((1, 1), jnp.float32)

    loss_ref[...] += part


def kernel(z, embedding):
    z_perm = jnp.transpose(z, (0, 2, 3, 1))
    flat_z = z_perm.reshape(-1, DIM)
    distances = (jnp.sum(flat_z ** 2, axis=1, keepdims=True)
                 - 2.0 * flat_z @ embedding
                 + jnp.sum(embedding ** 2, axis=0, keepdims=True))
    idx = jnp.argmin(distances, axis=1)

    idx_c = jnp.clip(idx.astype(jnp.int32), 0, NUM_E - 1)
    table = jnp.pad(embedding.T, ((0, 0), (0, DPAD - DIM)))
    q_flat = _sc_gather()(table, idx_c)

    q3 = q_flat.reshape(NB, HW, DPAD)
    z3 = z.reshape(NB, DIM, HW)
    qst, loss_sum = pl.pallas_call(
        _st_loss_body,
        grid=(NB,),
        in_specs=[pl.BlockSpec((1, HW, DPAD), lambda i: (i, 0, 0)),
                  pl.BlockSpec((1, DIM, HW), lambda i: (i, 0, 0))],
        out_specs=[pl.BlockSpec((1, DIM, HW), lambda i: (i, 0, 0)),
                   pl.BlockSpec((1, 1), lambda i: (0, 0))],
        out_shape=[jax.ShapeDtypeStruct((NB, DIM, HW), jnp.float32),
                   jax.ShapeDtypeStruct((1, 1), jnp.float32)],
    )(q3, z3)
    quantized_st = qst.reshape(NB, DIM, 64, 64)
    loss = loss_sum[0, 0] * (1.25 / (TOK * DIM))
    return quantized_st, loss
